# Initial kernel scaffold; baseline (speedup 1.0000x reference)
import jax
import jax.numpy as jnp
from jax.experimental import pallas as pl


def kernel(x, w, wout):
    raise NotImplementedError("write your pallas kernel here")


# speedup vs baseline: 2.5391x; 2.5391x over previous
import functools

import jax
import jax.numpy as jnp
from jax import lax
from jax.experimental import pallas as pl
from jax.experimental.pallas import tpu as pltpu

N_INPUTS = 4096
N_NEURONS = 4096
N_OUTPUTS = 1024
BATCH = 4096

TK1 = 1024
TB = 512


def _stage1_body(wout_ref, w_ref, t_ref):
    t_ref[...] = jnp.dot(wout_ref[...], w_ref[...],
                         preferred_element_type=jnp.float32)


def _stage2_body(t_ref, x_ref, out_ref):
    out_ref[...] = lax.dot_general(
        t_ref[...], x_ref[...],
        dimension_numbers=(((1,), (1,)), ((), ())),
        preferred_element_type=jnp.float32)


@jax.jit
def kernel(x, w, wout):
    t = pl.pallas_call(
        _stage1_body,
        grid=(N_INPUTS // TK1,),
        in_specs=[
            pl.BlockSpec((N_OUTPUTS, N_NEURONS), lambda i: (0, 0)),
            pl.BlockSpec((N_NEURONS, TK1), lambda i: (0, i)),
        ],
        out_specs=pl.BlockSpec((N_OUTPUTS, TK1), lambda i: (0, i)),
        out_shape=jax.ShapeDtypeStruct((N_OUTPUTS, N_INPUTS), jnp.float32),
    )(wout, w)

    out = pl.pallas_call(
        _stage2_body,
        grid=(BATCH // TB,),
        in_specs=[
            pl.BlockSpec((N_OUTPUTS, N_INPUTS), lambda i: (0, 0)),
            pl.BlockSpec((TB, N_INPUTS), lambda i: (i, 0)),
        ],
        out_specs=pl.BlockSpec((N_OUTPUTS, TB), lambda i: (0, i)),
        out_shape=jax.ShapeDtypeStruct((N_OUTPUTS, BATCH), jnp.float32),
    )(t, x)
    return out


# speedup vs baseline: 2.5858x; 1.0184x over previous
import functools

import jax
import jax.numpy as jnp
from jax import lax
from jax.experimental import pallas as pl
from jax.experimental.pallas import tpu as pltpu

N_INPUTS = 4096
N_NEURONS = 4096
N_OUTPUTS = 1024
BATCH = 4096

TK1 = 512
TB = 512


def _stage1_body(wout_ref, w_ref, t_ref):
    t_ref[...] = jnp.dot(wout_ref[...].astype(jnp.bfloat16),
                         w_ref[...].astype(jnp.bfloat16),
                         preferred_element_type=jnp.float32)


def _stage2_body(t_ref, x_ref, out_ref):
    out_ref[...] = lax.dot_general(
        t_ref[...].astype(jnp.bfloat16), x_ref[...].astype(jnp.bfloat16),
        dimension_numbers=(((1,), (1,)), ((), ())),
        preferred_element_type=jnp.float32)


@jax.jit
def kernel(x, w, wout):
    t = pl.pallas_call(
        _stage1_body,
        grid=(N_INPUTS // TK1,),
        in_specs=[
            pl.BlockSpec((N_OUTPUTS, N_NEURONS), lambda i: (0, 0)),
            pl.BlockSpec((N_NEURONS, TK1), lambda i: (0, i)),
        ],
        out_specs=pl.BlockSpec((N_OUTPUTS, TK1), lambda i: (0, i)),
        out_shape=jax.ShapeDtypeStruct((N_OUTPUTS, N_INPUTS), jnp.float32),
    )(wout, w)

    out = pl.pallas_call(
        _stage2_body,
        grid=(BATCH // TB,),
        in_specs=[
            pl.BlockSpec((N_OUTPUTS, N_INPUTS), lambda i: (0, 0)),
            pl.BlockSpec((TB, N_INPUTS), lambda i: (i, 0)),
        ],
        out_specs=pl.BlockSpec((N_OUTPUTS, TB), lambda i: (0, i)),
        out_shape=jax.ShapeDtypeStruct((N_OUTPUTS, BATCH), jnp.float32),
    )(t, x)
    return out
